# Initial kernel scaffold; baseline (speedup 1.0000x reference)
import jax
import jax.numpy as jnp
from jax.experimental import pallas as pl


def kernel(user_emb, item_emb, edge_index):
    raise NotImplementedError("write your pallas kernel here")


# speedup vs baseline: 9.9086x; 9.9086x over previous
import jax
import jax.numpy as jnp
from jax import lax
from jax.experimental import pallas as pl
from jax.experimental.pallas import tpu as pltpu
from jax.experimental.pallas import tpu_sc as plsc

NU = 25000
NI = 25000
N = NU + NI
D = 64
E = 800000
NUM_LAYERS = 3
ALPHA = 1.0 / (NUM_LAYERS + 1)

NC = 2
NS = 16
HALF = N // NC

RPT = 1568
ACC_R = NS * RPT
DUMMY = HALF

NRC = E // 128
SROWS = 16
NSC = NRC // SROWS
TAILR = NRC - NSC * SROWS

_mesh = plsc.VectorSubcoreMesh(
    core_axis_name="c", subcore_axis_name="s", num_cores=NC, num_subcores=NS)

_params = pltpu.CompilerParams(use_tc_tiling_on_sc=False)


def _local_cols(colbuf, lcbuf, j, base, dmy):
  for i in range(8):
    v = colbuf[j, pl.ds(i * 16, 16)]
    lcv = v - base
    ok = (lcv >= 0) & (lcv < HALF)
    lcbuf[j, pl.ds(i * 16, 16)] = jnp.where(ok, lcv, dmy)


def _segsum_body(y_hbm, row_hbm, col_hbm, zrows_hbm, out_hbm,
                 acc, rowbuf, colbuf, lcbuf, msgbuf):
  c = lax.axis_index("c")
  s = lax.axis_index("s")
  base = c * HALF
  dmy = DUMMY + s * 5

  pltpu.sync_copy(zrows_hbm, acc.at[pl.ds(s * RPT, RPT)])
  plsc.subcore_barrier()

  def chunk_body(k, carry):
    g = s + k * NS
    r0 = g * SROWS
    pltpu.sync_copy(row_hbm.at[pl.ds(r0, SROWS)], rowbuf)
    pltpu.sync_copy(col_hbm.at[pl.ds(r0, SROWS)], colbuf)
    for j in range(SROWS):
      _local_cols(colbuf, lcbuf, j, base, dmy)
    for j in range(SROWS):
      pltpu.sync_copy(y_hbm.at[rowbuf.at[j]], msgbuf)
      pltpu.sync_copy(msgbuf, acc.at[lcbuf.at[j]], add=True)
    return carry

  nsc = (NSC + NS - 1 - s) // NS
  lax.fori_loop(0, nsc, chunk_body, 0)

  @pl.when(s == NS - 1)
  def _():
    r0 = NSC * SROWS
    pltpu.sync_copy(row_hbm.at[pl.ds(r0, TAILR)], rowbuf.at[pl.ds(0, TAILR)])
    pltpu.sync_copy(col_hbm.at[pl.ds(r0, TAILR)], colbuf.at[pl.ds(0, TAILR)])
    for j in range(TAILR):
      _local_cols(colbuf, lcbuf, j, base, dmy)
      pltpu.sync_copy(y_hbm.at[rowbuf.at[j]], msgbuf)
      pltpu.sync_copy(msgbuf, acc.at[lcbuf.at[j]], add=True)

  plsc.subcore_barrier()
  pltpu.sync_copy(acc.at[pl.ds(s * RPT, RPT)],
                  out_hbm.at[c, pl.ds(s * RPT, RPT)])


_segsum = pl.kernel(
    _segsum_body,
    out_type=jax.ShapeDtypeStruct((NC, ACC_R, D), jnp.float32),
    mesh=_mesh,
    compiler_params=_params,
    scratch_types=[
        pltpu.VMEM_SHARED((ACC_R, D), jnp.float32),
        pltpu.VMEM((SROWS, 128), jnp.int32),
        pltpu.VMEM((SROWS, 128), jnp.int32),
        pltpu.VMEM((SROWS, 128), jnp.int32),
        pltpu.VMEM((128, D), jnp.float32),
    ],
)


def _deg_body(col_hbm, zrows_hbm, ones_hbm, out_hbm,
              acc, colbuf, lcbuf, onesbuf):
  c = lax.axis_index("c")
  s = lax.axis_index("s")
  base = c * HALF
  dmy = DUMMY + s * 5

  pltpu.sync_copy(zrows_hbm, acc.at[pl.ds(s * RPT, RPT)])
  pltpu.sync_copy(ones_hbm, onesbuf)
  plsc.subcore_barrier()

  def chunk_body(k, carry):
    g = s + k * NS
    r0 = g * SROWS
    pltpu.sync_copy(col_hbm.at[pl.ds(r0, SROWS)], colbuf)
    for j in range(SROWS):
      _local_cols(colbuf, lcbuf, j, base, dmy)
    for j in range(SROWS):
      pltpu.sync_copy(onesbuf, acc.at[lcbuf.at[j]], add=True)
    return carry

  nsc = (NSC + NS - 1 - s) // NS
  lax.fori_loop(0, nsc, chunk_body, 0)

  @pl.when(s == NS - 1)
  def _():
    r0 = NSC * SROWS
    pltpu.sync_copy(col_hbm.at[pl.ds(r0, TAILR)], colbuf.at[pl.ds(0, TAILR)])
    for j in range(TAILR):
      _local_cols(colbuf, lcbuf, j, base, dmy)
      pltpu.sync_copy(onesbuf, acc.at[lcbuf.at[j]], add=True)

  plsc.subcore_barrier()
  pltpu.sync_copy(acc.at[pl.ds(s * RPT, RPT)],
                  out_hbm.at[c, pl.ds(s * RPT, RPT)])


_deg = pl.kernel(
    _deg_body,
    out_type=jax.ShapeDtypeStruct((NC, ACC_R, 16), jnp.float32),
    mesh=_mesh,
    compiler_params=_params,
    scratch_types=[
        pltpu.VMEM_SHARED((ACC_R, 16), jnp.float32),
        pltpu.VMEM((SROWS, 128), jnp.int32),
        pltpu.VMEM((SROWS, 128), jnp.int32),
        pltpu.VMEM((128, 16), jnp.float32),
    ],
)


_BR = 1000


def _dinv_of(deg):
  return jnp.where(deg > 0, lax.rsqrt(jnp.maximum(deg, 1e-12)), 0.0)


def _scale_first_body(x_ref, deg_ref, y_ref, out_ref):
  x = x_ref[...]
  dinv = _dinv_of(deg_ref[...])
  y_ref[...] = x * dinv
  out_ref[...] = x * ALPHA


def _scale_first(x0, deg):
  return pl.pallas_call(
      _scale_first_body,
      grid=(N // _BR,),
      in_specs=[
          pl.BlockSpec((_BR, D), lambda b: (b, 0)),
          pl.BlockSpec((_BR, 1), lambda b: (b, 0)),
      ],
      out_specs=[
          pl.BlockSpec((_BR, D), lambda b: (b, 0)),
          pl.BlockSpec((_BR, D), lambda b: (b, 0)),
      ],
      out_shape=[
          jax.ShapeDtypeStruct((N, D), jnp.float32),
          jax.ShapeDtypeStruct((N, D), jnp.float32),
      ],
  )(x0, deg)


def _scale_mid_body(acc_ref, deg_ref, prev_ref, y_ref, out_ref):
  a = acc_ref[0]
  dinv = _dinv_of(deg_ref[...])
  x = a * dinv
  y_ref[...] = x * dinv
  out_ref[...] = prev_ref[...] + x * ALPHA


_PB = HALF // _BR


def _scale_mid(acc_planes, deg, prev):
  return pl.pallas_call(
      _scale_mid_body,
      grid=(NC, _PB),
      in_specs=[
          pl.BlockSpec((1, _BR, D), lambda p, b: (p, b, 0)),
          pl.BlockSpec((_BR, 1), lambda p, b: (p * _PB + b, 0)),
          pl.BlockSpec((_BR, D), lambda p, b: (p * _PB + b, 0)),
      ],
      out_specs=[
          pl.BlockSpec((_BR, D), lambda p, b: (p * _PB + b, 0)),
          pl.BlockSpec((_BR, D), lambda p, b: (p * _PB + b, 0)),
      ],
      out_shape=[
          jax.ShapeDtypeStruct((N, D), jnp.float32),
          jax.ShapeDtypeStruct((N, D), jnp.float32),
      ],
  )(acc_planes, deg, prev)


def kernel(user_emb, item_emb, edge_index):
  x0 = jnp.concatenate([user_emb, item_emb], axis=0)
  row2 = edge_index[0].astype(jnp.int32).reshape(NRC, 128)
  col2 = edge_index[1].astype(jnp.int32).reshape(NRC, 128)
  zr64 = jnp.zeros((RPT, D), jnp.float32)
  zr16 = jnp.zeros((RPT, 16), jnp.float32)
  ones16 = jnp.ones((128, 16), jnp.float32)

  deg_planes = _deg(col2, zr16, ones16)
  deg = jnp.concatenate(
      [deg_planes[0, :HALF, 0], deg_planes[1, :HALF, 0]])[:, None]

  y, out = _scale_first(x0, deg)
  for _ in range(NUM_LAYERS):
    acc_planes = _segsum(y, row2, col2, zr64)
    y, out = _scale_mid(acc_planes, deg, out)

  return out[:NU], out[NU:]


# speedup vs baseline: 11.3341x; 1.1439x over previous
import jax
import jax.numpy as jnp
from jax import lax
from jax.experimental import pallas as pl
from jax.experimental.pallas import tpu as pltpu
from jax.experimental.pallas import tpu_sc as plsc

NU = 25000
NI = 25000
N = NU + NI
D = 64
E = 800000
NUM_LAYERS = 3
ALPHA = 1.0 / (NUM_LAYERS + 1)

NC = 2
NS = 16
HALF = N // NC

RPT = 1664
ACC_R = NS * RPT
DUMMY = HALF
NBLK = RPT // 128

NRC = E // 128
SROWS = 16
NSC = NRC // SROWS
TAILR = NRC - NSC * SROWS

_mesh = plsc.VectorSubcoreMesh(
    core_axis_name="c", subcore_axis_name="s", num_cores=NC, num_subcores=NS)

_params = pltpu.CompilerParams(use_tc_tiling_on_sc=False)


def _local_cols(colbuf, lcbuf, j, base, dmy):
  for i in range(8):
    v = colbuf[j, pl.ds(i * 16, 16)]
    lcv = v - base
    ok = (lcv >= 0) & (lcv < HALF)
    lcbuf[j, pl.ds(i * 16, 16)] = jnp.where(ok, lcv, dmy)


def _segsum_body(y_hbm, row_hbm, col_hbm, d2_hbm, d1a_hbm, prev_hbm,
                 ynext_hbm, out_hbm,
                 acc, rowbuf, colbuf, lcbuf, msgA, msgB, semA, semB):
  c = lax.axis_index("c")
  s = lax.axis_index("s")
  base = c * HALF
  dmy = DUMMY + s * 100
  zvec = jnp.zeros((16,), jnp.float32)

  def zrow(r, carry):
    for t in range(4):
      msgA[r, pl.ds(t * 16, 16)] = zvec
    return carry
  lax.fori_loop(0, 128, zrow, 0)

  def zblk(b, carry):
    pltpu.sync_copy(msgA, acc.at[pl.ds(s * RPT + b * 128, 128)])
    return carry
  lax.fori_loop(0, NBLK, zblk, 0)
  plsc.subcore_barrier()

  bufs = (msgA, msgB)
  sems = (semA, semB)

  def chunk_body(k, carry):
    g = s + k * NS
    r0 = g * SROWS
    pltpu.sync_copy(row_hbm.at[pl.ds(r0, SROWS)], rowbuf)
    pltpu.sync_copy(col_hbm.at[pl.ds(r0, SROWS)], colbuf)
    for j in range(SROWS):
      _local_cols(colbuf, lcbuf, j, base, dmy)
    cps = [None] * SROWS
    cps[0] = pltpu.make_async_copy(y_hbm.at[rowbuf.at[0]], bufs[0], sems[0])
    cps[0].start()
    for j in range(SROWS):
      cps[j].wait()
      if j + 1 < SROWS:
        cps[j + 1] = pltpu.make_async_copy(
            y_hbm.at[rowbuf.at[j + 1]], bufs[(j + 1) % 2], sems[(j + 1) % 2])
        cps[j + 1].start()
      pltpu.sync_copy(bufs[j % 2], acc.at[lcbuf.at[j]], add=True)
    return carry

  nsc = (NSC + NS - 1 - s) // NS
  lax.fori_loop(0, nsc, chunk_body, 0)

  @pl.when(s == NS - 1)
  def _():
    r0 = NSC * SROWS
    pltpu.sync_copy(row_hbm.at[pl.ds(r0, TAILR)], rowbuf.at[pl.ds(0, TAILR)])
    pltpu.sync_copy(col_hbm.at[pl.ds(r0, TAILR)], colbuf.at[pl.ds(0, TAILR)])
    for j in range(TAILR):
      _local_cols(colbuf, lcbuf, j, base, dmy)
      pltpu.sync_copy(y_hbm.at[rowbuf.at[j]], msgA)
      pltpu.sync_copy(msgA, acc.at[lcbuf.at[j]], add=True)

  plsc.subcore_barrier()

  def wb_block(l0, g0, nrows):
    pltpu.sync_copy(acc.at[pl.ds(l0, nrows)], msgA.at[pl.ds(0, nrows)])
    pltpu.sync_copy(d2_hbm.at[pl.ds(g0, nrows)], msgB.at[pl.ds(0, nrows)])

    def mul_rows(r, carry):
      for t in range(4):
        sl = pl.ds(t * 16, 16)
        msgB[r, sl] = msgA[r, sl] * msgB[r, sl]
      return carry
    lax.fori_loop(0, nrows, mul_rows, 0)
    pltpu.sync_copy(msgB.at[pl.ds(0, nrows)], ynext_hbm.at[pl.ds(g0, nrows)])

    pltpu.sync_copy(d1a_hbm.at[pl.ds(g0, nrows)], msgB.at[pl.ds(0, nrows)])
    lax.fori_loop(0, nrows, mul_rows, 0)
    pltpu.sync_copy(prev_hbm.at[pl.ds(g0, nrows)], msgA.at[pl.ds(0, nrows)])

    def add_rows(r, carry):
      for t in range(4):
        sl = pl.ds(t * 16, 16)
        msgA[r, sl] = msgA[r, sl] + msgB[r, sl]
      return carry
    lax.fori_loop(0, nrows, add_rows, 0)
    pltpu.sync_copy(msgA.at[pl.ds(0, nrows)], out_hbm.at[pl.ds(g0, nrows)])

  @pl.when(s < NS - 1)
  def _():
    def wb(b, carry):
      l0 = s * RPT + b * 128
      wb_block(l0, base + l0, 128)
      return carry
    lax.fori_loop(0, NBLK, wb, 0)

  @pl.when(s == NS - 1)
  def _():
    wb_block((NS - 1) * RPT, base + (NS - 1) * RPT, HALF - (NS - 1) * RPT)


_segsum = pl.kernel(
    _segsum_body,
    out_type=(
        jax.ShapeDtypeStruct((N, D), jnp.float32),
        jax.ShapeDtypeStruct((N, D), jnp.float32),
    ),
    mesh=_mesh,
    compiler_params=_params,
    scratch_types=[
        pltpu.VMEM_SHARED((ACC_R, D), jnp.float32),
        pltpu.VMEM((SROWS, 128), jnp.int32),
        pltpu.VMEM((SROWS, 128), jnp.int32),
        pltpu.VMEM((SROWS, 128), jnp.int32),
        pltpu.VMEM((128, D), jnp.float32),
        pltpu.VMEM((128, D), jnp.float32),
        pltpu.SemaphoreType.DMA,
        pltpu.SemaphoreType.DMA,
    ],
)


def _deg_body(col_hbm, out_hbm, acc, colbuf, lcbuf, onesbuf, zbuf):
  c = lax.axis_index("c")
  s = lax.axis_index("s")
  base = c * HALF
  dmy = DUMMY + s * 100
  ovec = jnp.ones((16,), jnp.float32)
  zvec = jnp.zeros((16,), jnp.float32)

  def fill(r, carry):
    onesbuf[r, pl.ds(0, 16)] = ovec
    zbuf[r, pl.ds(0, 16)] = zvec
    return carry
  lax.fori_loop(0, 128, fill, 0)

  def zblk(b, carry):
    pltpu.sync_copy(zbuf, acc.at[pl.ds(s * RPT + b * 128, 128)])
    return carry
  lax.fori_loop(0, NBLK, zblk, 0)
  plsc.subcore_barrier()

  def chunk_body(k, carry):
    g = s + k * NS
    r0 = g * SROWS
    pltpu.sync_copy(col_hbm.at[pl.ds(r0, SROWS)], colbuf)
    for j in range(SROWS):
      _local_cols(colbuf, lcbuf, j, base, dmy)
    for j in range(SROWS):
      pltpu.sync_copy(onesbuf, acc.at[lcbuf.at[j]], add=True)
    return carry

  nsc = (NSC + NS - 1 - s) // NS
  lax.fori_loop(0, nsc, chunk_body, 0)

  @pl.when(s == NS - 1)
  def _():
    r0 = NSC * SROWS
    pltpu.sync_copy(col_hbm.at[pl.ds(r0, TAILR)], colbuf.at[pl.ds(0, TAILR)])
    for j in range(TAILR):
      _local_cols(colbuf, lcbuf, j, base, dmy)
      pltpu.sync_copy(onesbuf, acc.at[lcbuf.at[j]], add=True)

  plsc.subcore_barrier()
  pltpu.sync_copy(acc.at[pl.ds(s * RPT, RPT)],
                  out_hbm.at[c, pl.ds(s * RPT, RPT)])


_deg = pl.kernel(
    _deg_body,
    out_type=jax.ShapeDtypeStruct((NC, ACC_R, 16), jnp.float32),
    mesh=_mesh,
    compiler_params=_params,
    scratch_types=[
        pltpu.VMEM_SHARED((ACC_R, 16), jnp.float32),
        pltpu.VMEM((SROWS, 128), jnp.int32),
        pltpu.VMEM((SROWS, 128), jnp.int32),
        pltpu.VMEM((128, 16), jnp.float32),
        pltpu.VMEM((128, 16), jnp.float32),
    ],
)


_BR = 1000


def _dinv_of(deg):
  return jnp.where(deg > 0, lax.rsqrt(jnp.maximum(deg, 1e-12)), 0.0)


def _prep_body(deg_ref, d2_ref, d1a_ref):
  dinv = _dinv_of(deg_ref[...])
  one = jnp.ones((_BR, D), jnp.float32)
  d2_ref[...] = (dinv * dinv) * one
  d1a_ref[...] = (dinv * ALPHA) * one


def _prep(deg):
  return pl.pallas_call(
      _prep_body,
      grid=(N // _BR,),
      in_specs=[pl.BlockSpec((_BR, 1), lambda b: (b, 0))],
      out_specs=[
          pl.BlockSpec((_BR, D), lambda b: (b, 0)),
          pl.BlockSpec((_BR, D), lambda b: (b, 0)),
      ],
      out_shape=[
          jax.ShapeDtypeStruct((N, D), jnp.float32),
          jax.ShapeDtypeStruct((N, D), jnp.float32),
      ],
  )(deg)


def _scale_first_body(x_ref, deg_ref, y_ref, out_ref):
  x = x_ref[...]
  dinv = _dinv_of(deg_ref[...])
  y_ref[...] = x * dinv
  out_ref[...] = x * ALPHA


def _scale_first(x0, deg):
  return pl.pallas_call(
      _scale_first_body,
      grid=(N // _BR,),
      in_specs=[
          pl.BlockSpec((_BR, D), lambda b: (b, 0)),
          pl.BlockSpec((_BR, 1), lambda b: (b, 0)),
      ],
      out_specs=[
          pl.BlockSpec((_BR, D), lambda b: (b, 0)),
          pl.BlockSpec((_BR, D), lambda b: (b, 0)),
      ],
      out_shape=[
          jax.ShapeDtypeStruct((N, D), jnp.float32),
          jax.ShapeDtypeStruct((N, D), jnp.float32),
      ],
  )(x0, deg)


def kernel(user_emb, item_emb, edge_index):
  x0 = jnp.concatenate([user_emb, item_emb], axis=0)
  row2 = edge_index[0].astype(jnp.int32).reshape(NRC, 128)
  col2 = edge_index[1].astype(jnp.int32).reshape(NRC, 128)

  deg_planes = _deg(col2)
  deg = jnp.concatenate(
      [deg_planes[0, :HALF, 0], deg_planes[1, :HALF, 0]])[:, None]

  d2, d1a = _prep(deg)
  y, out = _scale_first(x0, deg)
  for _ in range(NUM_LAYERS):
    y, out = _segsum(y, row2, col2, d2, d1a, out)

  return out[:NU], out[NU:]


# speedup vs baseline: 11.3470x; 1.0011x over previous
import jax
import jax.numpy as jnp
from jax import lax
from jax.experimental import pallas as pl
from jax.experimental.pallas import tpu as pltpu
from jax.experimental.pallas import tpu_sc as plsc

NU = 25000
NI = 25000
N = NU + NI
D = 64
E = 800000
NUM_LAYERS = 3
ALPHA = 1.0 / (NUM_LAYERS + 1)

NC = 2
NS = 16
HALF = N // NC

RPT = 1664
ACC_R = NS * RPT
DUMMY = HALF
NBLK = RPT // 128

NRC = E // 128
SROWS = 16
NSC = NRC // SROWS
TAILR = NRC - NSC * SROWS

_mesh = plsc.VectorSubcoreMesh(
    core_axis_name="c", subcore_axis_name="s", num_cores=NC, num_subcores=NS)

_params = pltpu.CompilerParams(use_tc_tiling_on_sc=False)


def _local_cols(colbuf, lcbuf, j, base, dmy):
  for i in range(8):
    v = colbuf[j, pl.ds(i * 16, 16)]
    lcv = v - base
    ok = (lcv >= 0) & (lcv < HALF)
    lcbuf[j, pl.ds(i * 16, 16)] = jnp.where(ok, lcv, dmy)


def _segsum_body(y_hbm, row_hbm, col_hbm, d2_hbm, d1a_hbm, prev_hbm,
                 ynext_hbm, out_hbm,
                 acc, rowbuf, colbuf, lcbuf, msgA, msgB,
                 semA, semB, ssemA, ssemB):
  c = lax.axis_index("c")
  s = lax.axis_index("s")
  base = c * HALF
  dmy = DUMMY + s * 100
  zvec = jnp.zeros((16,), jnp.float32)

  def zrow(r, carry):
    for t in range(4):
      msgA[r, pl.ds(t * 16, 16)] = zvec
    return carry
  lax.fori_loop(0, 128, zrow, 0)

  def zblk(b, carry):
    pltpu.sync_copy(msgA, acc.at[pl.ds(s * RPT + b * 128, 128)])
    return carry
  lax.fori_loop(0, NBLK, zblk, 0)
  plsc.subcore_barrier()

  bufs = (msgA, msgB)
  sems = (semA, semB)
  ssems = (ssemA, ssemB)

  def chunk_body(k, carry):
    g = s + k * NS
    r0 = g * SROWS
    pltpu.sync_copy(row_hbm.at[pl.ds(r0, SROWS)], rowbuf)
    pltpu.sync_copy(col_hbm.at[pl.ds(r0, SROWS)], colbuf)
    for j in range(SROWS):
      _local_cols(colbuf, lcbuf, j, base, dmy)
    gcp = [
        pltpu.make_async_copy(
            y_hbm.at[rowbuf.at[j]], bufs[j % 2], sems[j % 2])
        for j in range(SROWS)
    ]
    scp = [
        pltpu.make_async_copy(
            bufs[j % 2], acc.at[lcbuf.at[j]], ssems[j % 2])
        for j in range(SROWS)
    ]
    gcp[0].start()
    for j in range(SROWS):
      gcp[j].wait()
      if j >= 1:
        scp[j - 1].wait()
      if j + 1 < SROWS:
        gcp[j + 1].start()
      scp[j].start(add=True)
    scp[SROWS - 1].wait()
    return carry

  nsc = (NSC + NS - 1 - s) // NS
  lax.fori_loop(0, nsc, chunk_body, 0)

  @pl.when(s == NS - 1)
  def _():
    r0 = NSC * SROWS
    pltpu.sync_copy(row_hbm.at[pl.ds(r0, TAILR)], rowbuf.at[pl.ds(0, TAILR)])
    pltpu.sync_copy(col_hbm.at[pl.ds(r0, TAILR)], colbuf.at[pl.ds(0, TAILR)])
    for j in range(TAILR):
      _local_cols(colbuf, lcbuf, j, base, dmy)
      pltpu.sync_copy(y_hbm.at[rowbuf.at[j]], msgA)
      pltpu.sync_copy(msgA, acc.at[lcbuf.at[j]], add=True)

  plsc.subcore_barrier()

  def wb_block(l0, g0, nrows):
    pltpu.sync_copy(acc.at[pl.ds(l0, nrows)], msgA.at[pl.ds(0, nrows)])
    pltpu.sync_copy(d2_hbm.at[pl.ds(g0, nrows)], msgB.at[pl.ds(0, nrows)])

    def mul_rows(r, carry):
      for t in range(4):
        sl = pl.ds(t * 16, 16)
        msgB[r, sl] = msgA[r, sl] * msgB[r, sl]
      return carry
    lax.fori_loop(0, nrows, mul_rows, 0)
    pltpu.sync_copy(msgB.at[pl.ds(0, nrows)], ynext_hbm.at[pl.ds(g0, nrows)])

    pltpu.sync_copy(d1a_hbm.at[pl.ds(g0, nrows)], msgB.at[pl.ds(0, nrows)])
    lax.fori_loop(0, nrows, mul_rows, 0)
    pltpu.sync_copy(prev_hbm.at[pl.ds(g0, nrows)], msgA.at[pl.ds(0, nrows)])

    def add_rows(r, carry):
      for t in range(4):
        sl = pl.ds(t * 16, 16)
        msgA[r, sl] = msgA[r, sl] + msgB[r, sl]
      return carry
    lax.fori_loop(0, nrows, add_rows, 0)
    pltpu.sync_copy(msgA.at[pl.ds(0, nrows)], out_hbm.at[pl.ds(g0, nrows)])

  @pl.when(s < NS - 1)
  def _():
    def wb(b, carry):
      l0 = s * RPT + b * 128
      wb_block(l0, base + l0, 128)
      return carry
    lax.fori_loop(0, NBLK, wb, 0)

  @pl.when(s == NS - 1)
  def _():
    wb_block((NS - 1) * RPT, base + (NS - 1) * RPT, HALF - (NS - 1) * RPT)


_segsum = pl.kernel(
    _segsum_body,
    out_type=(
        jax.ShapeDtypeStruct((N, D), jnp.float32),
        jax.ShapeDtypeStruct((N, D), jnp.float32),
    ),
    mesh=_mesh,
    compiler_params=_params,
    scratch_types=[
        pltpu.VMEM_SHARED((ACC_R, D), jnp.float32),
        pltpu.VMEM((SROWS, 128), jnp.int32),
        pltpu.VMEM((SROWS, 128), jnp.int32),
        pltpu.VMEM((SROWS, 128), jnp.int32),
        pltpu.VMEM((128, D), jnp.float32),
        pltpu.VMEM((128, D), jnp.float32),
        pltpu.SemaphoreType.DMA,
        pltpu.SemaphoreType.DMA,
        pltpu.SemaphoreType.DMA,
        pltpu.SemaphoreType.DMA,
    ],
)


def _deg_body(col_hbm, out_hbm, acc, colbuf, lcbuf, onesbuf, zbuf, ssem):
  c = lax.axis_index("c")
  s = lax.axis_index("s")
  base = c * HALF
  dmy = DUMMY + s * 100
  ovec = jnp.ones((16,), jnp.float32)
  zvec = jnp.zeros((16,), jnp.float32)

  def fill(r, carry):
    onesbuf[r, pl.ds(0, 16)] = ovec
    zbuf[r, pl.ds(0, 16)] = zvec
    return carry
  lax.fori_loop(0, 128, fill, 0)

  def zblk(b, carry):
    pltpu.sync_copy(zbuf, acc.at[pl.ds(s * RPT + b * 128, 128)])
    return carry
  lax.fori_loop(0, NBLK, zblk, 0)
  plsc.subcore_barrier()

  def chunk_body(k, carry):
    g = s + k * NS
    r0 = g * SROWS
    pltpu.sync_copy(col_hbm.at[pl.ds(r0, SROWS)], colbuf)
    for j in range(SROWS):
      _local_cols(colbuf, lcbuf, j, base, dmy)
    scp = [
        pltpu.make_async_copy(onesbuf, acc.at[lcbuf.at[j]], ssem)
        for j in range(SROWS)
    ]
    for j in range(SROWS):
      scp[j].start(add=True)
    for j in range(SROWS):
      scp[j].wait()
    return carry

  nsc = (NSC + NS - 1 - s) // NS
  lax.fori_loop(0, nsc, chunk_body, 0)

  @pl.when(s == NS - 1)
  def _():
    r0 = NSC * SROWS
    pltpu.sync_copy(col_hbm.at[pl.ds(r0, TAILR)], colbuf.at[pl.ds(0, TAILR)])
    for j in range(TAILR):
      _local_cols(colbuf, lcbuf, j, base, dmy)
      pltpu.sync_copy(onesbuf, acc.at[lcbuf.at[j]], add=True)

  plsc.subcore_barrier()
  pltpu.sync_copy(acc.at[pl.ds(s * RPT, RPT)],
                  out_hbm.at[c, pl.ds(s * RPT, RPT)])


_deg = pl.kernel(
    _deg_body,
    out_type=jax.ShapeDtypeStruct((NC, ACC_R, 16), jnp.float32),
    mesh=_mesh,
    compiler_params=_params,
    scratch_types=[
        pltpu.VMEM_SHARED((ACC_R, 16), jnp.float32),
        pltpu.VMEM((SROWS, 128), jnp.int32),
        pltpu.VMEM((SROWS, 128), jnp.int32),
        pltpu.VMEM((128, 16), jnp.float32),
        pltpu.VMEM((128, 16), jnp.float32),
        pltpu.SemaphoreType.DMA,
    ],
)


_BR = 1000


def _dinv_of(deg):
  return jnp.where(deg > 0, lax.rsqrt(jnp.maximum(deg, 1e-12)), 0.0)


def _prep_body(deg_ref, d2_ref, d1a_ref):
  dinv = _dinv_of(deg_ref[...])
  one = jnp.ones((_BR, D), jnp.float32)
  d2_ref[...] = (dinv * dinv) * one
  d1a_ref[...] = (dinv * ALPHA) * one


def _prep(deg):
  return pl.pallas_call(
      _prep_body,
      grid=(N // _BR,),
      in_specs=[pl.BlockSpec((_BR, 1), lambda b: (b, 0))],
      out_specs=[
          pl.BlockSpec((_BR, D), lambda b: (b, 0)),
          pl.BlockSpec((_BR, D), lambda b: (b, 0)),
      ],
      out_shape=[
          jax.ShapeDtypeStruct((N, D), jnp.float32),
          jax.ShapeDtypeStruct((N, D), jnp.float32),
      ],
  )(deg)


def _scale_first_body(x_ref, deg_ref, y_ref, out_ref):
  x = x_ref[...]
  dinv = _dinv_of(deg_ref[...])
  y_ref[...] = x * dinv
  out_ref[...] = x * ALPHA


def _scale_first(x0, deg):
  return pl.pallas_call(
      _scale_first_body,
      grid=(N // _BR,),
      in_specs=[
          pl.BlockSpec((_BR, D), lambda b: (b, 0)),
          pl.BlockSpec((_BR, 1), lambda b: (b, 0)),
      ],
      out_specs=[
          pl.BlockSpec((_BR, D), lambda b: (b, 0)),
          pl.BlockSpec((_BR, D), lambda b: (b, 0)),
      ],
      out_shape=[
          jax.ShapeDtypeStruct((N, D), jnp.float32),
          jax.ShapeDtypeStruct((N, D), jnp.float32),
      ],
  )(x0, deg)


def kernel(user_emb, item_emb, edge_index):
  x0 = jnp.concatenate([user_emb, item_emb], axis=0)
  row2 = edge_index[0].astype(jnp.int32).reshape(NRC, 128)
  col2 = edge_index[1].astype(jnp.int32).reshape(NRC, 128)

  deg_planes = _deg(col2)
  deg = jnp.concatenate(
      [deg_planes[0, :HALF, 0], deg_planes[1, :HALF, 0]])[:, None]

  d2, d1a = _prep(deg)
  y, out = _scale_first(x0, deg)
  for _ in range(NUM_LAYERS):
    y, out = _segsum(y, row2, col2, d2, d1a, out)

  return out[:NU], out[NU:]


# speedup vs baseline: 12.0762x; 1.0643x over previous
import jax
import jax.numpy as jnp
from jax import lax
from jax.experimental import pallas as pl
from jax.experimental.pallas import tpu as pltpu
from jax.experimental.pallas import tpu_sc as plsc

NU = 25000
NI = 25000
N = NU + NI
D = 64
E = 800000
NUM_LAYERS = 3
ALPHA = 1.0 / (NUM_LAYERS + 1)

NC = 2
NS = 16
HALF = N // NC

RPT = 1664
ACC_R = NS * RPT
DUMMY = HALF
NBLK = RPT // 128

NRC = E // 128
SROWS = 8
NSC = NRC // SROWS
TAILR = NRC - NSC * SROWS
STG = 1280

_mesh = plsc.VectorSubcoreMesh(
    core_axis_name="c", subcore_axis_name="s", num_cores=NC, num_subcores=NS)

_params = pltpu.CompilerParams(
    use_tc_tiling_on_sc=False, needs_layout_passes=False)


def _compact_rows(rowbuf, colbuf, stgr, stgl, nrows, base, cur):
  for j in range(nrows):
    for i in range(8):
      sl = pl.ds(i * 16, 16)
      cv = colbuf[j, sl]
      lcv = cv - base
      ok = (lcv >= 0) & (lcv < HALF)
      oki = jnp.where(ok, 1, 0)
      pfx = plsc.cumsum(oki)
      idx = cur + (pfx - oki)
      plsc.store_scatter(stgr, [idx], rowbuf[j, sl], mask=ok)
      plsc.store_scatter(stgl, [idx], lcv, mask=ok)
      cur = cur + jnp.sum(oki)
  return cur


def _vcopy128(src, src_off, dst):
  for i in range(8):
    dst[pl.ds(i * 16, 16)] = src[pl.ds(src_off + i * 16, 16)]


def _segsum_body(y_hbm, row_hbm, col_hbm, d2_hbm, d1a_hbm, prev_hbm,
                 ynext_hbm, out_hbm,
                 acc, rowbuf, colbuf, stgr, stgl, ridx, lidx, msgA, msgB):
  c = lax.axis_index("c")
  s = lax.axis_index("s")
  base = c * HALF
  dmy = DUMMY + s * 100
  zvec = jnp.zeros((16,), jnp.float32)

  def zrow(r, carry):
    for t in range(4):
      msgA[r, pl.ds(t * 16, 16)] = zvec
    return carry
  lax.fori_loop(0, 128, zrow, 0)

  def zblk(b, carry):
    pltpu.sync_copy(msgA, acc.at[pl.ds(s * RPT + b * 128, 128)])
    return carry
  lax.fori_loop(0, NBLK, zblk, 0)
  plsc.subcore_barrier()

  def fire_slabs(nf):
    def fire(b, carry):
      _vcopy128(stgr, b * 128, ridx)
      _vcopy128(stgl, b * 128, lidx)
      pltpu.sync_copy(y_hbm.at[ridx], msgA)
      pltpu.sync_copy(msgA, acc.at[lidx], add=True)
      return carry
    lax.fori_loop(0, nf, fire, 0)

  def chunk_body(k, cur):
    g = s + k * NS
    r0 = g * SROWS
    pltpu.sync_copy(row_hbm.at[pl.ds(r0, SROWS)], rowbuf)
    pltpu.sync_copy(col_hbm.at[pl.ds(r0, SROWS)], colbuf)
    cur = _compact_rows(rowbuf, colbuf, stgr, stgl, SROWS, base, cur)
    nf = cur >> 7
    fire_slabs(nf)
    @pl.when(nf > 0)
    def _():
      _vcopy128(stgr, nf * 128, ridx)
      _vcopy128(ridx, 0, stgr)
      _vcopy128(stgl, nf * 128, lidx)
      _vcopy128(lidx, 0, stgl)
    return cur - nf * 128

  nsc = (NSC + NS - 1 - s) // NS
  cur = lax.fori_loop(0, nsc, chunk_body, jnp.int32(0))

  def tail_fn(cur):
    r0 = NSC * SROWS
    pltpu.sync_copy(row_hbm.at[pl.ds(r0, TAILR)], rowbuf.at[pl.ds(0, TAILR)])
    pltpu.sync_copy(col_hbm.at[pl.ds(r0, TAILR)], colbuf.at[pl.ds(0, TAILR)])
    return _compact_rows(rowbuf, colbuf, stgr, stgl, TAILR, base, cur)

  cur = lax.cond(s == NS - 1, tail_fn, lambda cur: cur, cur)

  for i in range(8):
    stgr[pl.ds(cur + i * 16, 16)] = jnp.zeros((16,), jnp.int32)
    stgl[pl.ds(cur + i * 16, 16)] = jnp.full((16,), 1, jnp.int32) * dmy
  fire_slabs((cur + 127) >> 7)

  plsc.subcore_barrier()

  def wb_block(l0, g0, nrows):
    pltpu.sync_copy(acc.at[pl.ds(l0, nrows)], msgA.at[pl.ds(0, nrows)])
    pltpu.sync_copy(d2_hbm.at[pl.ds(g0, nrows)], msgB.at[pl.ds(0, nrows)])

    def mul_rows(r, carry):
      for t in range(4):
        sl = pl.ds(t * 16, 16)
        msgB[r, sl] = msgA[r, sl] * msgB[r, sl]
      return carry
    lax.fori_loop(0, nrows, mul_rows, 0)
    pltpu.sync_copy(msgB.at[pl.ds(0, nrows)], ynext_hbm.at[pl.ds(g0, nrows)])

    pltpu.sync_copy(d1a_hbm.at[pl.ds(g0, nrows)], msgB.at[pl.ds(0, nrows)])
    lax.fori_loop(0, nrows, mul_rows, 0)
    pltpu.sync_copy(prev_hbm.at[pl.ds(g0, nrows)], msgA.at[pl.ds(0, nrows)])

    def add_rows(r, carry):
      for t in range(4):
        sl = pl.ds(t * 16, 16)
        msgA[r, sl] = msgA[r, sl] + msgB[r, sl]
      return carry
    lax.fori_loop(0, nrows, add_rows, 0)
    pltpu.sync_copy(msgA.at[pl.ds(0, nrows)], out_hbm.at[pl.ds(g0, nrows)])

  @pl.when(s < NS - 1)
  def _():
    def wb(b, carry):
      l0 = s * RPT + b * 128
      wb_block(l0, base + l0, 128)
      return carry
    lax.fori_loop(0, NBLK, wb, 0)

  @pl.when(s == NS - 1)
  def _():
    wb_block((NS - 1) * RPT, base + (NS - 1) * RPT, HALF - (NS - 1) * RPT)


_segsum = pl.kernel(
    _segsum_body,
    out_type=(
        jax.ShapeDtypeStruct((N, D), jnp.float32),
        jax.ShapeDtypeStruct((N, D), jnp.float32),
    ),
    mesh=_mesh,
    compiler_params=_params,
    scratch_types=[
        pltpu.VMEM_SHARED((ACC_R, D), jnp.float32),
        pltpu.VMEM((SROWS, 128), jnp.int32),
        pltpu.VMEM((SROWS, 128), jnp.int32),
        pltpu.VMEM((STG,), jnp.int32),
        pltpu.VMEM((STG,), jnp.int32),
        pltpu.VMEM((128,), jnp.int32),
        pltpu.VMEM((128,), jnp.int32),
        pltpu.VMEM((128, D), jnp.float32),
        pltpu.VMEM((128, D), jnp.float32),
    ],
)


def _deg_compact(colbuf, stgl, nrows, base, cur):
  for j in range(nrows):
    for i in range(8):
      cv = colbuf[j, pl.ds(i * 16, 16)]
      lcv = cv - base
      ok = (lcv >= 0) & (lcv < HALF)
      oki = jnp.where(ok, 1, 0)
      pfx = plsc.cumsum(oki)
      idx = cur + (pfx - oki)
      plsc.store_scatter(stgl, [idx], lcv, mask=ok)
      cur = cur + jnp.sum(oki)
  return cur


def _deg_body(col_hbm, out_hbm, acc, colbuf, stgl, lidx, onesbuf, zbuf):
  c = lax.axis_index("c")
  s = lax.axis_index("s")
  base = c * HALF
  dmy = DUMMY + s * 100
  ovec = jnp.ones((16,), jnp.float32)
  zvec = jnp.zeros((16,), jnp.float32)

  def fill(r, carry):
    onesbuf[r, pl.ds(0, 16)] = ovec
    zbuf[r, pl.ds(0, 16)] = zvec
    return carry
  lax.fori_loop(0, 128, fill, 0)

  def zblk(b, carry):
    pltpu.sync_copy(zbuf, acc.at[pl.ds(s * RPT + b * 128, 128)])
    return carry
  lax.fori_loop(0, NBLK, zblk, 0)
  plsc.subcore_barrier()

  def fire_slabs(nf):
    def fire(b, carry):
      _vcopy128(stgl, b * 128, lidx)
      pltpu.sync_copy(onesbuf, acc.at[lidx], add=True)
      return carry
    lax.fori_loop(0, nf, fire, 0)

  def chunk_body(k, cur):
    g = s + k * NS
    r0 = g * SROWS
    pltpu.sync_copy(col_hbm.at[pl.ds(r0, SROWS)], colbuf)
    cur = _deg_compact(colbuf, stgl, SROWS, base, cur)
    nf = cur >> 7
    fire_slabs(nf)
    @pl.when(nf > 0)
    def _():
      _vcopy128(stgl, nf * 128, lidx)
      _vcopy128(lidx, 0, stgl)
    return cur - nf * 128

  nsc = (NSC + NS - 1 - s) // NS
  cur = lax.fori_loop(0, nsc, chunk_body, jnp.int32(0))

  def tail_fn(cur):
    r0 = NSC * SROWS
    pltpu.sync_copy(col_hbm.at[pl.ds(r0, TAILR)], colbuf.at[pl.ds(0, TAILR)])
    return _deg_compact(colbuf, stgl, TAILR, base, cur)

  cur = lax.cond(s == NS - 1, tail_fn, lambda cur: cur, cur)

  for i in range(8):
    stgl[pl.ds(cur + i * 16, 16)] = jnp.full((16,), 1, jnp.int32) * dmy
  fire_slabs((cur + 127) >> 7)

  plsc.subcore_barrier()
  pltpu.sync_copy(acc.at[pl.ds(s * RPT, RPT)],
                  out_hbm.at[c, pl.ds(s * RPT, RPT)])


_deg = pl.kernel(
    _deg_body,
    out_type=jax.ShapeDtypeStruct((NC, ACC_R, 16), jnp.float32),
    mesh=_mesh,
    compiler_params=_params,
    scratch_types=[
        pltpu.VMEM_SHARED((ACC_R, 16), jnp.float32),
        pltpu.VMEM((SROWS, 128), jnp.int32),
        pltpu.VMEM((STG,), jnp.int32),
        pltpu.VMEM((128,), jnp.int32),
        pltpu.VMEM((128, 16), jnp.float32),
        pltpu.VMEM((128, 16), jnp.float32),
    ],
)


_BR = 1000


def _dinv_of(deg):
  return jnp.where(deg > 0, lax.rsqrt(jnp.maximum(deg, 1e-12)), 0.0)


def _prep_body(deg_ref, d2_ref, d1a_ref):
  dinv = _dinv_of(deg_ref[...])
  one = jnp.ones((_BR, D), jnp.float32)
  d2_ref[...] = (dinv * dinv) * one
  d1a_ref[...] = (dinv * ALPHA) * one


def _prep(deg):
  return pl.pallas_call(
      _prep_body,
      grid=(N // _BR,),
      in_specs=[pl.BlockSpec((_BR, 1), lambda b: (b, 0))],
      out_specs=[
          pl.BlockSpec((_BR, D), lambda b: (b, 0)),
          pl.BlockSpec((_BR, D), lambda b: (b, 0)),
      ],
      out_shape=[
          jax.ShapeDtypeStruct((N, D), jnp.float32),
          jax.ShapeDtypeStruct((N, D), jnp.float32),
      ],
  )(deg)


def _scale_first_body(x_ref, deg_ref, y_ref, out_ref):
  x = x_ref[...]
  dinv = _dinv_of(deg_ref[...])
  y_ref[...] = x * dinv
  out_ref[...] = x * ALPHA


def _scale_first(x0, deg):
  return pl.pallas_call(
      _scale_first_body,
      grid=(N // _BR,),
      in_specs=[
          pl.BlockSpec((_BR, D), lambda b: (b, 0)),
          pl.BlockSpec((_BR, 1), lambda b: (b, 0)),
      ],
      out_specs=[
          pl.BlockSpec((_BR, D), lambda b: (b, 0)),
          pl.BlockSpec((_BR, D), lambda b: (b, 0)),
      ],
      out_shape=[
          jax.ShapeDtypeStruct((N, D), jnp.float32),
          jax.ShapeDtypeStruct((N, D), jnp.float32),
      ],
  )(x0, deg)


def kernel(user_emb, item_emb, edge_index):
  x0 = jnp.concatenate([user_emb, item_emb], axis=0)
  row2 = edge_index[0].astype(jnp.int32).reshape(NRC, 128)
  col2 = edge_index[1].astype(jnp.int32).reshape(NRC, 128)

  deg_planes = _deg(col2)
  deg = jnp.concatenate(
      [deg_planes[0, :HALF, 0], deg_planes[1, :HALF, 0]])[:, None]

  d2, d1a = _prep(deg)
  y, out = _scale_first(x0, deg)
  for _ in range(NUM_LAYERS):
    y, out = _segsum(y, row2, col2, d2, d1a, out)

  return out[:NU], out[NU:]


# speedup vs baseline: 14.2718x; 1.1818x over previous
import jax
import jax.numpy as jnp
from jax import lax
from jax.experimental import pallas as pl
from jax.experimental.pallas import tpu as pltpu
from jax.experimental.pallas import tpu_sc as plsc

NU = 25000
NI = 25000
N = NU + NI
D = 64
E = 800000
NUM_LAYERS = 3
ALPHA = 1.0 / (NUM_LAYERS + 1)

NC = 2
NS = 16
HALF = N // NC

RPT = 1664
ACC_R = NS * RPT
DUMMY = HALF
NBLK = RPT // 128

NRC = E // 128
SROWS = 8
NSC = NRC // SROWS
TAILR = NRC - NSC * SROWS
STG = 1280

_mesh = plsc.VectorSubcoreMesh(
    core_axis_name="c", subcore_axis_name="s", num_cores=NC, num_subcores=NS)

_params = pltpu.CompilerParams(
    use_tc_tiling_on_sc=False, needs_layout_passes=False)


def _compact_rows(rowbuf, colbuf, stgr, stgl, nrows, base, cur):
  for j in range(nrows):
    for i in range(8):
      sl = pl.ds(i * 16, 16)
      cv = colbuf[j, sl]
      lcv = cv - base
      ok = (lcv >= 0) & (lcv < HALF)
      oki = jnp.where(ok, 1, 0)
      pfx = plsc.cumsum(oki)
      idx = cur + (pfx - oki)
      plsc.store_scatter(stgr, [idx], rowbuf[j, sl], mask=ok)
      plsc.store_scatter(stgl, [idx], lcv, mask=ok)
      cur = cur + jnp.sum(oki)
  return cur


def _vcopy128(src, src_off, dst):
  for i in range(8):
    dst[pl.ds(i * 16, 16)] = src[pl.ds(src_off + i * 16, 16)]


def _segsum_body(y_hbm, row_hbm, col_hbm, d2_hbm, d1a_hbm, prev_hbm,
                 ynext_hbm, out_hbm,
                 acc, rowbuf, colbuf, stgr, stgl, ridxA, lidxA, ridxB, lidxB,
                 msgA, msgB, gsemA, gsemB, ssemA, ssemB):
  c = lax.axis_index("c")
  s = lax.axis_index("s")
  base = c * HALF
  dmy = DUMMY + s * 100
  zvec = jnp.zeros((16,), jnp.float32)

  def zrow(r, carry):
    for t in range(4):
      msgA[r, pl.ds(t * 16, 16)] = zvec
    return carry
  lax.fori_loop(0, 128, zrow, 0)

  def zblk(b, carry):
    pltpu.sync_copy(msgA, acc.at[pl.ds(s * RPT + b * 128, 128)])
    return carry
  lax.fori_loop(0, NBLK, zblk, 0)
  plsc.subcore_barrier()

  msg = (msgA, msgB)
  gsem = (gsemA, gsemB)
  ssem = (ssemA, ssemB)
  ridx = (ridxA, ridxB)
  lidx = (lidxA, lidxB)

  def fire_slabs(nf):
    for b in range(9):
      p = b % 2
      if b < 8:
        @pl.when(b < nf)
        def _(b=b, p=p):
          if b >= 2:
            pltpu.make_async_copy(msg[p], acc.at[lidx[p]], ssem[p]).wait()
          _vcopy128(stgr, b * 128, ridx[p])
          _vcopy128(stgl, b * 128, lidx[p])
          pltpu.make_async_copy(y_hbm.at[ridx[p]], msg[p], gsem[p]).start()
      if b >= 1:
        @pl.when(b - 1 < nf)
        def _(b=b):
          q = (b - 1) % 2
          pltpu.make_async_copy(y_hbm.at[ridx[q]], msg[q], gsem[q]).wait()
          pltpu.make_async_copy(
              msg[q], acc.at[lidx[q]], ssem[q]).start(add=True)
    for b in range(8):
      @pl.when((b >= nf - 2) & (b < nf))
      def _(b=b):
        pltpu.make_async_copy(
            msg[b % 2], acc.at[lidx[b % 2]], ssem[b % 2]).wait()

  def chunk_body(k, cur):
    g = s + k * NS
    r0 = g * SROWS
    pltpu.sync_copy(row_hbm.at[pl.ds(r0, SROWS)], rowbuf)
    pltpu.sync_copy(col_hbm.at[pl.ds(r0, SROWS)], colbuf)
    cur = _compact_rows(rowbuf, colbuf, stgr, stgl, SROWS, base, cur)
    nf = cur >> 7
    fire_slabs(nf)
    @pl.when(nf > 0)
    def _():
      _vcopy128(stgr, nf * 128, ridxA)
      _vcopy128(ridxA, 0, stgr)
      _vcopy128(stgl, nf * 128, lidxA)
      _vcopy128(lidxA, 0, stgl)
    return cur - nf * 128

  nsc = (NSC + NS - 1 - s) // NS
  cur = lax.fori_loop(0, nsc, chunk_body, jnp.int32(0))

  def tail_fn(cur):
    r0 = NSC * SROWS
    pltpu.sync_copy(row_hbm.at[pl.ds(r0, TAILR)], rowbuf.at[pl.ds(0, TAILR)])
    pltpu.sync_copy(col_hbm.at[pl.ds(r0, TAILR)], colbuf.at[pl.ds(0, TAILR)])
    return _compact_rows(rowbuf, colbuf, stgr, stgl, TAILR, base, cur)

  cur = lax.cond(s == NS - 1, tail_fn, lambda cur: cur, cur)

  for i in range(8):
    stgr[pl.ds(cur + i * 16, 16)] = jnp.zeros((16,), jnp.int32)
    stgl[pl.ds(cur + i * 16, 16)] = jnp.full((16,), 1, jnp.int32) * dmy
  fire_slabs((cur + 127) >> 7)

  plsc.subcore_barrier()

  def wb_block(l0, g0, nrows):
    pltpu.sync_copy(acc.at[pl.ds(l0, nrows)], msgA.at[pl.ds(0, nrows)])
    pltpu.sync_copy(d2_hbm.at[pl.ds(g0, nrows)], msgB.at[pl.ds(0, nrows)])

    def mul_rows(r, carry):
      for t in range(4):
        sl = pl.ds(t * 16, 16)
        msgB[r, sl] = msgA[r, sl] * msgB[r, sl]
      return carry
    lax.fori_loop(0, nrows, mul_rows, 0)
    pltpu.sync_copy(msgB.at[pl.ds(0, nrows)], ynext_hbm.at[pl.ds(g0, nrows)])

    pltpu.sync_copy(d1a_hbm.at[pl.ds(g0, nrows)], msgB.at[pl.ds(0, nrows)])
    lax.fori_loop(0, nrows, mul_rows, 0)
    pltpu.sync_copy(prev_hbm.at[pl.ds(g0, nrows)], msgA.at[pl.ds(0, nrows)])

    def add_rows(r, carry):
      for t in range(4):
        sl = pl.ds(t * 16, 16)
        msgA[r, sl] = msgA[r, sl] + msgB[r, sl]
      return carry
    lax.fori_loop(0, nrows, add_rows, 0)
    pltpu.sync_copy(msgA.at[pl.ds(0, nrows)], out_hbm.at[pl.ds(g0, nrows)])

  @pl.when(s < NS - 1)
  def _():
    def wb(b, carry):
      l0 = s * RPT + b * 128
      wb_block(l0, base + l0, 128)
      return carry
    lax.fori_loop(0, NBLK, wb, 0)

  @pl.when(s == NS - 1)
  def _():
    wb_block((NS - 1) * RPT, base + (NS - 1) * RPT, HALF - (NS - 1) * RPT)


_segsum = pl.kernel(
    _segsum_body,
    out_type=(
        jax.ShapeDtypeStruct((N, D), jnp.float32),
        jax.ShapeDtypeStruct((N, D), jnp.float32),
    ),
    mesh=_mesh,
    compiler_params=_params,
    scratch_types=[
        pltpu.VMEM_SHARED((ACC_R, D), jnp.float32),
        pltpu.VMEM((SROWS, 128), jnp.int32),
        pltpu.VMEM((SROWS, 128), jnp.int32),
        pltpu.VMEM((STG,), jnp.int32),
        pltpu.VMEM((STG,), jnp.int32),
        pltpu.VMEM((128,), jnp.int32),
        pltpu.VMEM((128,), jnp.int32),
        pltpu.VMEM((128,), jnp.int32),
        pltpu.VMEM((128,), jnp.int32),
        pltpu.VMEM((128, D), jnp.float32),
        pltpu.VMEM((128, D), jnp.float32),
        pltpu.SemaphoreType.DMA,
        pltpu.SemaphoreType.DMA,
        pltpu.SemaphoreType.DMA,
        pltpu.SemaphoreType.DMA,
    ],
)


def _deg_compact(colbuf, stgl, nrows, base, cur):
  for j in range(nrows):
    for i in range(8):
      cv = colbuf[j, pl.ds(i * 16, 16)]
      lcv = cv - base
      ok = (lcv >= 0) & (lcv < HALF)
      oki = jnp.where(ok, 1, 0)
      pfx = plsc.cumsum(oki)
      idx = cur + (pfx - oki)
      plsc.store_scatter(stgl, [idx], lcv, mask=ok)
      cur = cur + jnp.sum(oki)
  return cur


def _deg_body(col_hbm, out_hbm, acc, colbuf, stgl, lidx, onesbuf, zbuf):
  c = lax.axis_index("c")
  s = lax.axis_index("s")
  base = c * HALF
  dmy = DUMMY + s * 100
  ovec = jnp.ones((16,), jnp.float32)
  zvec = jnp.zeros((16,), jnp.float32)

  def fill(r, carry):
    onesbuf[r, pl.ds(0, 16)] = ovec
    zbuf[r, pl.ds(0, 16)] = zvec
    return carry
  lax.fori_loop(0, 128, fill, 0)

  def zblk(b, carry):
    pltpu.sync_copy(zbuf, acc.at[pl.ds(s * RPT + b * 128, 128)])
    return carry
  lax.fori_loop(0, NBLK, zblk, 0)
  plsc.subcore_barrier()

  def fire_slabs(nf):
    def fire(b, carry):
      _vcopy128(stgl, b * 128, lidx)
      pltpu.sync_copy(onesbuf, acc.at[lidx], add=True)
      return carry
    lax.fori_loop(0, nf, fire, 0)

  def chunk_body(k, cur):
    g = s + k * NS
    r0 = g * SROWS
    pltpu.sync_copy(col_hbm.at[pl.ds(r0, SROWS)], colbuf)
    cur = _deg_compact(colbuf, stgl, SROWS, base, cur)
    nf = cur >> 7
    fire_slabs(nf)
    @pl.when(nf > 0)
    def _():
      _vcopy128(stgl, nf * 128, lidx)
      _vcopy128(lidx, 0, stgl)
    return cur - nf * 128

  nsc = (NSC + NS - 1 - s) // NS
  cur = lax.fori_loop(0, nsc, chunk_body, jnp.int32(0))

  def tail_fn(cur):
    r0 = NSC * SROWS
    pltpu.sync_copy(col_hbm.at[pl.ds(r0, TAILR)], colbuf.at[pl.ds(0, TAILR)])
    return _deg_compact(colbuf, stgl, TAILR, base, cur)

  cur = lax.cond(s == NS - 1, tail_fn, lambda cur: cur, cur)

  for i in range(8):
    stgl[pl.ds(cur + i * 16, 16)] = jnp.full((16,), 1, jnp.int32) * dmy
  fire_slabs((cur + 127) >> 7)

  plsc.subcore_barrier()
  pltpu.sync_copy(acc.at[pl.ds(s * RPT, RPT)],
                  out_hbm.at[c, pl.ds(s * RPT, RPT)])


_deg = pl.kernel(
    _deg_body,
    out_type=jax.ShapeDtypeStruct((NC, ACC_R, 16), jnp.float32),
    mesh=_mesh,
    compiler_params=_params,
    scratch_types=[
        pltpu.VMEM_SHARED((ACC_R, 16), jnp.float32),
        pltpu.VMEM((SROWS, 128), jnp.int32),
        pltpu.VMEM((STG,), jnp.int32),
        pltpu.VMEM((128,), jnp.int32),
        pltpu.VMEM((128, 16), jnp.float32),
        pltpu.VMEM((128, 16), jnp.float32),
    ],
)


_BR = 1000


def _dinv_of(deg):
  return jnp.where(deg > 0, lax.rsqrt(jnp.maximum(deg, 1e-12)), 0.0)


def _prep_body(deg_ref, d2_ref, d1a_ref):
  dinv = _dinv_of(deg_ref[...])
  one = jnp.ones((_BR, D), jnp.float32)
  d2_ref[...] = (dinv * dinv) * one
  d1a_ref[...] = (dinv * ALPHA) * one


def _prep(deg):
  return pl.pallas_call(
      _prep_body,
      grid=(N // _BR,),
      in_specs=[pl.BlockSpec((_BR, 1), lambda b: (b, 0))],
      out_specs=[
          pl.BlockSpec((_BR, D), lambda b: (b, 0)),
          pl.BlockSpec((_BR, D), lambda b: (b, 0)),
      ],
      out_shape=[
          jax.ShapeDtypeStruct((N, D), jnp.float32),
          jax.ShapeDtypeStruct((N, D), jnp.float32),
      ],
  )(deg)


def _scale_first_body(x_ref, deg_ref, y_ref, out_ref):
  x = x_ref[...]
  dinv = _dinv_of(deg_ref[...])
  y_ref[...] = x * dinv
  out_ref[...] = x * ALPHA


def _scale_first(x0, deg):
  return pl.pallas_call(
      _scale_first_body,
      grid=(N // _BR,),
      in_specs=[
          pl.BlockSpec((_BR, D), lambda b: (b, 0)),
          pl.BlockSpec((_BR, 1), lambda b: (b, 0)),
      ],
      out_specs=[
          pl.BlockSpec((_BR, D), lambda b: (b, 0)),
          pl.BlockSpec((_BR, D), lambda b: (b, 0)),
      ],
      out_shape=[
          jax.ShapeDtypeStruct((N, D), jnp.float32),
          jax.ShapeDtypeStruct((N, D), jnp.float32),
      ],
  )(x0, deg)


def kernel(user_emb, item_emb, edge_index):
  x0 = jnp.concatenate([user_emb, item_emb], axis=0)
  row2 = edge_index[0].astype(jnp.int32).reshape(NRC, 128)
  col2 = edge_index[1].astype(jnp.int32).reshape(NRC, 128)

  deg_planes = _deg(col2)
  deg = jnp.concatenate(
      [deg_planes[0, :HALF, 0], deg_planes[1, :HALF, 0]])[:, None]

  d2, d1a = _prep(deg)
  y, out = _scale_first(x0, deg)
  for _ in range(NUM_LAYERS):
    y, out = _segsum(y, row2, col2, d2, d1a, out)

  return out[:NU], out[NU:]


# speedup vs baseline: 14.7270x; 1.0319x over previous
import jax
import jax.numpy as jnp
from jax import lax
from jax.experimental import pallas as pl
from jax.experimental.pallas import tpu as pltpu
from jax.experimental.pallas import tpu_sc as plsc

NU = 25000
NI = 25000
N = NU + NI
D = 64
E = 800000
NUM_LAYERS = 3
ALPHA = 1.0 / (NUM_LAYERS + 1)

NC = 2
NS = 16
HALF = N // NC

RPT = 1664
ACC_R = NS * RPT
DUMMY = HALF
NBLK = RPT // 128

NRC = E // 128
SROWS = 8
NSC = NRC // SROWS
TAILR = NRC - NSC * SROWS
STG = 1280

_mesh = plsc.VectorSubcoreMesh(
    core_axis_name="c", subcore_axis_name="s", num_cores=NC, num_subcores=NS)

_params = pltpu.CompilerParams(
    use_tc_tiling_on_sc=False, needs_layout_passes=False)


def _compact_rows(rowbuf, colbuf, stgr, stgl, nrows, base, cur):
  for j in range(nrows):
    for i in range(8):
      sl = pl.ds(i * 16, 16)
      cv = colbuf[j, sl]
      lcv = cv - base
      ok = (lcv >= 0) & (lcv < HALF)
      oki = jnp.where(ok, 1, 0)
      pfx = plsc.cumsum(oki)
      idx = cur + (pfx - oki)
      plsc.store_scatter(stgr, [idx], rowbuf[j, sl], mask=ok)
      plsc.store_scatter(stgl, [idx], lcv, mask=ok)
      cur = cur + jnp.sum(oki)
  return cur


def _vcopy128(src, src_off, dst):
  for i in range(8):
    dst[pl.ds(i * 16, 16)] = src[pl.ds(src_off + i * 16, 16)]


def _segsum_body(y_hbm, row_hbm, col_hbm, d2_hbm, d1a_hbm, prev_hbm,
                 ynext_hbm, out_hbm,
                 acc, rowbuf, colbuf, stgr, stgl, ridxA, lidxA, ridxB, lidxB,
                 msgA, msgB, gsemA, gsemB, ssemA, ssemB):
  c = lax.axis_index("c")
  s = lax.axis_index("s")
  base = c * HALF
  dmy = DUMMY + s * 100
  zvec = jnp.zeros((16,), jnp.float32)

  def zrow(r, carry):
    for t in range(4):
      msgA[r, pl.ds(t * 16, 16)] = zvec
    return carry
  lax.fori_loop(0, 128, zrow, 0)

  def zblk(b, carry):
    pltpu.sync_copy(msgA, acc.at[pl.ds(s * RPT + b * 128, 128)])
    return carry
  lax.fori_loop(0, NBLK, zblk, 0)
  plsc.subcore_barrier()

  msg = (msgA, msgB)
  gsem = (gsemA, gsemB)
  ssem = (ssemA, ssemB)
  ridx = (ridxA, ridxB)
  lidx = (lidxA, lidxB)

  def fire_slabs(nf):
    for b in range(9):
      p = b % 2
      if b < 8:
        @pl.when(b < nf)
        def _(b=b, p=p):
          if b >= 2:
            pltpu.make_async_copy(msg[p], acc.at[lidx[p]], ssem[p]).wait()
          _vcopy128(stgr, b * 128, ridx[p])
          _vcopy128(stgl, b * 128, lidx[p])
          pltpu.make_async_copy(y_hbm.at[ridx[p]], msg[p], gsem[p]).start()
      if b >= 1:
        @pl.when(b - 1 < nf)
        def _(b=b):
          q = (b - 1) % 2
          pltpu.make_async_copy(y_hbm.at[ridx[q]], msg[q], gsem[q]).wait()
          pltpu.make_async_copy(
              msg[q], acc.at[lidx[q]], ssem[q]).start(add=True)
    for b in range(8):
      @pl.when((b >= nf - 2) & (b < nf))
      def _(b=b):
        pltpu.make_async_copy(
            msg[b % 2], acc.at[lidx[b % 2]], ssem[b % 2]).wait()

  def chunk_body(k, cur):
    g = s + k * NS
    r0 = g * SROWS
    pltpu.sync_copy(row_hbm.at[pl.ds(r0, SROWS)], rowbuf)
    pltpu.sync_copy(col_hbm.at[pl.ds(r0, SROWS)], colbuf)
    cur = _compact_rows(rowbuf, colbuf, stgr, stgl, SROWS, base, cur)
    nf = cur >> 7
    fire_slabs(nf)
    @pl.when(nf > 0)
    def _():
      _vcopy128(stgr, nf * 128, ridxA)
      _vcopy128(ridxA, 0, stgr)
      _vcopy128(stgl, nf * 128, lidxA)
      _vcopy128(lidxA, 0, stgl)
    return cur - nf * 128

  nsc = (NSC + NS - 1 - s) // NS
  cur = lax.fori_loop(0, nsc, chunk_body, jnp.int32(0))

  def tail_fn(cur):
    r0 = NSC * SROWS
    pltpu.sync_copy(row_hbm.at[pl.ds(r0, TAILR)], rowbuf.at[pl.ds(0, TAILR)])
    pltpu.sync_copy(col_hbm.at[pl.ds(r0, TAILR)], colbuf.at[pl.ds(0, TAILR)])
    return _compact_rows(rowbuf, colbuf, stgr, stgl, TAILR, base, cur)

  cur = lax.cond(s == NS - 1, tail_fn, lambda cur: cur, cur)

  for i in range(8):
    stgr[pl.ds(cur + i * 16, 16)] = jnp.zeros((16,), jnp.int32)
    stgl[pl.ds(cur + i * 16, 16)] = jnp.full((16,), 1, jnp.int32) * dmy
  fire_slabs((cur + 127) >> 7)

  plsc.subcore_barrier()

  def wb_block(l0, g0, nrows):
    pltpu.sync_copy(acc.at[pl.ds(l0, nrows)], msgA.at[pl.ds(0, nrows)])
    pltpu.sync_copy(d2_hbm.at[pl.ds(g0, nrows)], msgB.at[pl.ds(0, nrows)])

    def mul_rows(r, carry):
      for t in range(4):
        sl = pl.ds(t * 16, 16)
        msgB[r, sl] = msgA[r, sl] * msgB[r, sl]
      return carry
    lax.fori_loop(0, nrows, mul_rows, 0)
    pltpu.sync_copy(msgB.at[pl.ds(0, nrows)], ynext_hbm.at[pl.ds(g0, nrows)])

    pltpu.sync_copy(d1a_hbm.at[pl.ds(g0, nrows)], msgB.at[pl.ds(0, nrows)])
    lax.fori_loop(0, nrows, mul_rows, 0)
    pltpu.sync_copy(prev_hbm.at[pl.ds(g0, nrows)], msgA.at[pl.ds(0, nrows)])

    def add_rows(r, carry):
      for t in range(4):
        sl = pl.ds(t * 16, 16)
        msgA[r, sl] = msgA[r, sl] + msgB[r, sl]
      return carry
    lax.fori_loop(0, nrows, add_rows, 0)
    pltpu.sync_copy(msgA.at[pl.ds(0, nrows)], out_hbm.at[pl.ds(g0, nrows)])

  @pl.when(s < NS - 1)
  def _():
    def wb(b, carry):
      l0 = s * RPT + b * 128
      wb_block(l0, base + l0, 128)
      return carry
    lax.fori_loop(0, NBLK, wb, 0)

  @pl.when(s == NS - 1)
  def _():
    wb_block((NS - 1) * RPT, base + (NS - 1) * RPT, HALF - (NS - 1) * RPT)


_segsum = pl.kernel(
    _segsum_body,
    out_type=(
        jax.ShapeDtypeStruct((N, D), jnp.float32),
        jax.ShapeDtypeStruct((N, D), jnp.float32),
    ),
    mesh=_mesh,
    compiler_params=_params,
    scratch_types=[
        pltpu.VMEM_SHARED((ACC_R, D), jnp.float32),
        pltpu.VMEM((SROWS, 128), jnp.int32),
        pltpu.VMEM((SROWS, 128), jnp.int32),
        pltpu.VMEM((STG,), jnp.int32),
        pltpu.VMEM((STG,), jnp.int32),
        pltpu.VMEM((128,), jnp.int32),
        pltpu.VMEM((128,), jnp.int32),
        pltpu.VMEM((128,), jnp.int32),
        pltpu.VMEM((128,), jnp.int32),
        pltpu.VMEM((128, D), jnp.float32),
        pltpu.VMEM((128, D), jnp.float32),
        pltpu.SemaphoreType.DMA,
        pltpu.SemaphoreType.DMA,
        pltpu.SemaphoreType.DMA,
        pltpu.SemaphoreType.DMA,
    ],
)


def _deg_compact(colbuf, stgl, nrows, base, cur):
  for j in range(nrows):
    for i in range(8):
      cv = colbuf[j, pl.ds(i * 16, 16)]
      lcv = cv - base
      ok = (lcv >= 0) & (lcv < HALF)
      oki = jnp.where(ok, 1, 0)
      pfx = plsc.cumsum(oki)
      idx = cur + (pfx - oki)
      plsc.store_scatter(stgl, [idx], lcv, mask=ok)
      cur = cur + jnp.sum(oki)
  return cur


def _deg_body(col_hbm, out_hbm, acc, colbuf, stgl, lidxA, lidxB, onesbuf,
              zbuf, ssemA, ssemB):
  c = lax.axis_index("c")
  s = lax.axis_index("s")
  base = c * HALF
  dmy = DUMMY + s * 100
  ovec = jnp.ones((16,), jnp.float32)
  zvec = jnp.zeros((16,), jnp.float32)

  def fill(r, carry):
    onesbuf[r, pl.ds(0, 16)] = ovec
    zbuf[r, pl.ds(0, 16)] = zvec
    return carry
  lax.fori_loop(0, 128, fill, 0)

  def zblk(b, carry):
    pltpu.sync_copy(zbuf, acc.at[pl.ds(s * RPT + b * 128, 128)])
    return carry
  lax.fori_loop(0, NBLK, zblk, 0)
  plsc.subcore_barrier()

  lidx = (lidxA, lidxB)
  ssem = (ssemA, ssemB)

  def fire_slabs(nf):
    for b in range(8):
      p = b % 2
      @pl.when(b < nf)
      def _(b=b, p=p):
        if b >= 2:
          pltpu.make_async_copy(onesbuf, acc.at[lidx[p]], ssem[p]).wait()
        _vcopy128(stgl, b * 128, lidx[p])
        pltpu.make_async_copy(
            onesbuf, acc.at[lidx[p]], ssem[p]).start(add=True)
    for b in range(8):
      @pl.when((b >= nf - 2) & (b < nf))
      def _(b=b):
        pltpu.make_async_copy(
            onesbuf, acc.at[lidx[b % 2]], ssem[b % 2]).wait()

  def chunk_body(k, cur):
    g = s + k * NS
    r0 = g * SROWS
    pltpu.sync_copy(col_hbm.at[pl.ds(r0, SROWS)], colbuf)
    cur = _deg_compact(colbuf, stgl, SROWS, base, cur)
    nf = cur >> 7
    fire_slabs(nf)
    @pl.when(nf > 0)
    def _():
      _vcopy128(stgl, nf * 128, lidxA)
      _vcopy128(lidxA, 0, stgl)
    return cur - nf * 128

  nsc = (NSC + NS - 1 - s) // NS
  cur = lax.fori_loop(0, nsc, chunk_body, jnp.int32(0))

  def tail_fn(cur):
    r0 = NSC * SROWS
    pltpu.sync_copy(col_hbm.at[pl.ds(r0, TAILR)], colbuf.at[pl.ds(0, TAILR)])
    return _deg_compact(colbuf, stgl, TAILR, base, cur)

  cur = lax.cond(s == NS - 1, tail_fn, lambda cur: cur, cur)

  for i in range(8):
    stgl[pl.ds(cur + i * 16, 16)] = jnp.full((16,), 1, jnp.int32) * dmy
  fire_slabs((cur + 127) >> 7)

  plsc.subcore_barrier()
  pltpu.sync_copy(acc.at[pl.ds(s * RPT, RPT)],
                  out_hbm.at[c, pl.ds(s * RPT, RPT)])


_deg = pl.kernel(
    _deg_body,
    out_type=jax.ShapeDtypeStruct((NC, ACC_R, 16), jnp.float32),
    mesh=_mesh,
    compiler_params=_params,
    scratch_types=[
        pltpu.VMEM_SHARED((ACC_R, 16), jnp.float32),
        pltpu.VMEM((SROWS, 128), jnp.int32),
        pltpu.VMEM((STG,), jnp.int32),
        pltpu.VMEM((128,), jnp.int32),
        pltpu.VMEM((128,), jnp.int32),
        pltpu.VMEM((128, 16), jnp.float32),
        pltpu.VMEM((128, 16), jnp.float32),
        pltpu.SemaphoreType.DMA,
        pltpu.SemaphoreType.DMA,
    ],
)


_BR = 1000


def _dinv_of(deg):
  return jnp.where(deg > 0, lax.rsqrt(jnp.maximum(deg, 1e-12)), 0.0)


def _init_body(x_ref, deg_ref, y_ref, out_ref, d2_ref, d1a_ref):
  x = x_ref[...]
  dinv = _dinv_of(deg_ref[...])
  one = jnp.ones((_BR, D), jnp.float32)
  y_ref[...] = x * dinv
  out_ref[...] = x * ALPHA
  d2_ref[...] = (dinv * dinv) * one
  d1a_ref[...] = (dinv * ALPHA) * one


def _init(x0, deg):
  return pl.pallas_call(
      _init_body,
      grid=(N // _BR,),
      in_specs=[
          pl.BlockSpec((_BR, D), lambda b: (b, 0)),
          pl.BlockSpec((_BR, 1), lambda b: (b, 0)),
      ],
      out_specs=[pl.BlockSpec((_BR, D), lambda b: (b, 0))] * 4,
      out_shape=[jax.ShapeDtypeStruct((N, D), jnp.float32)] * 4,
  )(x0, deg)


def kernel(user_emb, item_emb, edge_index):
  x0 = jnp.concatenate([user_emb, item_emb], axis=0)
  row2 = edge_index[0].astype(jnp.int32).reshape(NRC, 128)
  col2 = edge_index[1].astype(jnp.int32).reshape(NRC, 128)

  deg_planes = _deg(col2)
  deg = jnp.concatenate(
      [deg_planes[0, :HALF, 0], deg_planes[1, :HALF, 0]])[:, None]

  y, out, d2, d1a = _init(x0, deg)
  for _ in range(NUM_LAYERS):
    y, out = _segsum(y, row2, col2, d2, d1a, out)

  return out[:NU], out[NU:]


# speedup vs baseline: 15.0043x; 1.0188x over previous
import functools

import jax
import jax.numpy as jnp
from jax import lax
from jax.experimental import pallas as pl
from jax.experimental.pallas import tpu as pltpu
from jax.experimental.pallas import tpu_sc as plsc

NU = 25000
NI = 25000
N = NU + NI
D = 64
E = 800000
NUM_LAYERS = 3
ALPHA = 1.0 / (NUM_LAYERS + 1)

NC = 2
NS = 16
HALF = N // NC

RPT = 1664
ACC_R = NS * RPT
DUMMY = HALF
NBLK = RPT // 128

NRC = E // 128
SROWS = 8
NSC = NRC // SROWS
TAILR = NRC - NSC * SROWS
STG = 1280

_mesh = plsc.VectorSubcoreMesh(
    core_axis_name="c", subcore_axis_name="s", num_cores=NC, num_subcores=NS)

_params = pltpu.CompilerParams(
    use_tc_tiling_on_sc=False, needs_layout_passes=False)


def _compact_rows(rowbuf, colbuf, stgr, stgl, nrows, base, cur):
  for j in range(nrows):
    for i in range(8):
      sl = pl.ds(i * 16, 16)
      cv = colbuf[j, sl]
      lcv = cv - base
      ok = (lcv >= 0) & (lcv < HALF)
      oki = jnp.where(ok, 1, 0)
      pfx = plsc.cumsum(oki)
      idx = cur + (pfx - oki)
      plsc.store_scatter(stgr, [idx], rowbuf[j, sl], mask=ok)
      plsc.store_scatter(stgl, [idx], lcv, mask=ok)
      cur = cur + jnp.sum(oki)
  return cur


def _vcopy128(src, src_off, dst):
  for i in range(8):
    dst[pl.ds(i * 16, 16)] = src[pl.ds(src_off + i * 16, 16)]


def _segsum_body(y_hbm, row_hbm, col_hbm, d2_hbm, d1a_hbm, prev_hbm,
                 ynext_hbm, out_hbm,
                 acc, rowbuf, colbuf, stgr, stgl, ridxA, lidxA, ridxB, lidxB,
                 msgA, msgB, gsemA, gsemB, ssemA, ssemB, *, last=False):
  c = lax.axis_index("c")
  s = lax.axis_index("s")
  base = c * HALF
  dmy = DUMMY + s * 100
  zvec = jnp.zeros((16,), jnp.float32)

  def zrow(r, carry):
    for t in range(4):
      msgA[r, pl.ds(t * 16, 16)] = zvec
    return carry
  lax.fori_loop(0, 128, zrow, 0)

  def zblk(b, carry):
    pltpu.sync_copy(msgA, acc.at[pl.ds(s * RPT + b * 128, 128)])
    return carry
  lax.fori_loop(0, NBLK, zblk, 0)
  plsc.subcore_barrier()

  msg = (msgA, msgB)
  gsem = (gsemA, gsemB)
  ssem = (ssemA, ssemB)
  ridx = (ridxA, ridxB)
  lidx = (lidxA, lidxB)

  def fire_slabs(nf):
    for b in range(9):
      p = b % 2
      if b < 8:
        @pl.when(b < nf)
        def _(b=b, p=p):
          if b >= 2:
            pltpu.make_async_copy(msg[p], acc.at[lidx[p]], ssem[p]).wait()
          _vcopy128(stgr, b * 128, ridx[p])
          _vcopy128(stgl, b * 128, lidx[p])
          pltpu.make_async_copy(y_hbm.at[ridx[p]], msg[p], gsem[p]).start()
      if b >= 1:
        @pl.when(b - 1 < nf)
        def _(b=b):
          q = (b - 1) % 2
          pltpu.make_async_copy(y_hbm.at[ridx[q]], msg[q], gsem[q]).wait()
          pltpu.make_async_copy(
              msg[q], acc.at[lidx[q]], ssem[q]).start(add=True)
    for b in range(8):
      @pl.when((b >= nf - 2) & (b < nf))
      def _(b=b):
        pltpu.make_async_copy(
            msg[b % 2], acc.at[lidx[b % 2]], ssem[b % 2]).wait()

  def chunk_body(k, cur):
    g = s + k * NS
    r0 = g * SROWS
    pltpu.sync_copy(row_hbm.at[pl.ds(r0, SROWS)], rowbuf)
    pltpu.sync_copy(col_hbm.at[pl.ds(r0, SROWS)], colbuf)
    cur = _compact_rows(rowbuf, colbuf, stgr, stgl, SROWS, base, cur)
    nf = cur >> 7
    fire_slabs(nf)
    @pl.when(nf > 0)
    def _():
      _vcopy128(stgr, nf * 128, ridxA)
      _vcopy128(ridxA, 0, stgr)
      _vcopy128(stgl, nf * 128, lidxA)
      _vcopy128(lidxA, 0, stgl)
    return cur - nf * 128

  nsc = (NSC + NS - 1 - s) // NS
  cur = lax.fori_loop(0, nsc, chunk_body, jnp.int32(0))

  def tail_fn(cur):
    r0 = NSC * SROWS
    pltpu.sync_copy(row_hbm.at[pl.ds(r0, TAILR)], rowbuf.at[pl.ds(0, TAILR)])
    pltpu.sync_copy(col_hbm.at[pl.ds(r0, TAILR)], colbuf.at[pl.ds(0, TAILR)])
    return _compact_rows(rowbuf, colbuf, stgr, stgl, TAILR, base, cur)

  cur = lax.cond(s == NS - 1, tail_fn, lambda cur: cur, cur)

  for i in range(8):
    stgr[pl.ds(cur + i * 16, 16)] = jnp.zeros((16,), jnp.int32)
    stgl[pl.ds(cur + i * 16, 16)] = jnp.full((16,), 1, jnp.int32) * dmy
  fire_slabs((cur + 127) >> 7)

  plsc.subcore_barrier()

  def wb_block(l0, g0, nrows):
    pltpu.sync_copy(acc.at[pl.ds(l0, nrows)], msgA.at[pl.ds(0, nrows)])

    def mul_rows(r, carry):
      for t in range(4):
        sl = pl.ds(t * 16, 16)
        msgB[r, sl] = msgA[r, sl] * msgB[r, sl]
      return carry

    if not last:
      pltpu.sync_copy(d2_hbm.at[pl.ds(g0, nrows)], msgB.at[pl.ds(0, nrows)])
      lax.fori_loop(0, nrows, mul_rows, 0)
      pltpu.sync_copy(
          msgB.at[pl.ds(0, nrows)], ynext_hbm.at[pl.ds(g0, nrows)])

    pltpu.sync_copy(d1a_hbm.at[pl.ds(g0, nrows)], msgB.at[pl.ds(0, nrows)])
    lax.fori_loop(0, nrows, mul_rows, 0)
    pltpu.sync_copy(prev_hbm.at[pl.ds(g0, nrows)], msgA.at[pl.ds(0, nrows)])

    def add_rows(r, carry):
      for t in range(4):
        sl = pl.ds(t * 16, 16)
        msgA[r, sl] = msgA[r, sl] + msgB[r, sl]
      return carry
    lax.fori_loop(0, nrows, add_rows, 0)
    pltpu.sync_copy(msgA.at[pl.ds(0, nrows)], out_hbm.at[pl.ds(g0, nrows)])

  @pl.when(s < NS - 1)
  def _():
    def wb(b, carry):
      l0 = s * RPT + b * 128
      wb_block(l0, base + l0, 128)
      return carry
    lax.fori_loop(0, NBLK, wb, 0)

  @pl.when(s == NS - 1)
  def _():
    wb_block((NS - 1) * RPT, base + (NS - 1) * RPT, HALF - (NS - 1) * RPT)


_SEG_SCRATCH = [
    pltpu.VMEM_SHARED((ACC_R, D), jnp.float32),
    pltpu.VMEM((SROWS, 128), jnp.int32),
    pltpu.VMEM((SROWS, 128), jnp.int32),
    pltpu.VMEM((STG,), jnp.int32),
    pltpu.VMEM((STG,), jnp.int32),
    pltpu.VMEM((128,), jnp.int32),
    pltpu.VMEM((128,), jnp.int32),
    pltpu.VMEM((128,), jnp.int32),
    pltpu.VMEM((128,), jnp.int32),
    pltpu.VMEM((128, D), jnp.float32),
    pltpu.VMEM((128, D), jnp.float32),
    pltpu.SemaphoreType.DMA,
    pltpu.SemaphoreType.DMA,
    pltpu.SemaphoreType.DMA,
    pltpu.SemaphoreType.DMA,
]

_segsum_last = pl.kernel(
    functools.partial(_segsum_body, last=True),
    out_type=(
        jax.ShapeDtypeStruct((N, D), jnp.float32),
        jax.ShapeDtypeStruct((N, D), jnp.float32),
    ),
    mesh=_mesh,
    compiler_params=_params,
    scratch_types=_SEG_SCRATCH,
)

_segsum = pl.kernel(
    _segsum_body,
    out_type=(
        jax.ShapeDtypeStruct((N, D), jnp.float32),
        jax.ShapeDtypeStruct((N, D), jnp.float32),
    ),
    mesh=_mesh,
    compiler_params=_params,
    scratch_types=_SEG_SCRATCH,
)


def _deg_compact(colbuf, stgl, nrows, base, cur):
  for j in range(nrows):
    for i in range(8):
      cv = colbuf[j, pl.ds(i * 16, 16)]
      lcv = cv - base
      ok = (lcv >= 0) & (lcv < HALF)
      oki = jnp.where(ok, 1, 0)
      pfx = plsc.cumsum(oki)
      idx = cur + (pfx - oki)
      plsc.store_scatter(stgl, [idx], lcv, mask=ok)
      cur = cur + jnp.sum(oki)
  return cur


def _deg_body(col_hbm, out_hbm, acc, colbuf, stgl, lidxA, lidxB, onesbuf,
              zbuf, ssemA, ssemB):
  c = lax.axis_index("c")
  s = lax.axis_index("s")
  base = c * HALF
  dmy = DUMMY + s * 100
  ovec = jnp.ones((16,), jnp.float32)
  zvec = jnp.zeros((16,), jnp.float32)

  def fill(r, carry):
    onesbuf[r, pl.ds(0, 16)] = ovec
    zbuf[r, pl.ds(0, 16)] = zvec
    return carry
  lax.fori_loop(0, 128, fill, 0)

  def zblk(b, carry):
    pltpu.sync_copy(zbuf, acc.at[pl.ds(s * RPT + b * 128, 128)])
    return carry
  lax.fori_loop(0, NBLK, zblk, 0)
  plsc.subcore_barrier()

  lidx = (lidxA, lidxB)
  ssem = (ssemA, ssemB)

  def fire_slabs(nf):
    for b in range(8):
      p = b % 2
      @pl.when(b < nf)
      def _(b=b, p=p):
        if b >= 2:
          pltpu.make_async_copy(onesbuf, acc.at[lidx[p]], ssem[p]).wait()
        _vcopy128(stgl, b * 128, lidx[p])
        pltpu.make_async_copy(
            onesbuf, acc.at[lidx[p]], ssem[p]).start(add=True)
    for b in range(8):
      @pl.when((b >= nf - 2) & (b < nf))
      def _(b=b):
        pltpu.make_async_copy(
            onesbuf, acc.at[lidx[b % 2]], ssem[b % 2]).wait()

  def chunk_body(k, cur):
    g = s + k * NS
    r0 = g * SROWS
    pltpu.sync_copy(col_hbm.at[pl.ds(r0, SROWS)], colbuf)
    cur = _deg_compact(colbuf, stgl, SROWS, base, cur)
    nf = cur >> 7
    fire_slabs(nf)
    @pl.when(nf > 0)
    def _():
      _vcopy128(stgl, nf * 128, lidxA)
      _vcopy128(lidxA, 0, stgl)
    return cur - nf * 128

  nsc = (NSC + NS - 1 - s) // NS
  cur = lax.fori_loop(0, nsc, chunk_body, jnp.int32(0))

  def tail_fn(cur):
    r0 = NSC * SROWS
    pltpu.sync_copy(col_hbm.at[pl.ds(r0, TAILR)], colbuf.at[pl.ds(0, TAILR)])
    return _deg_compact(colbuf, stgl, TAILR, base, cur)

  cur = lax.cond(s == NS - 1, tail_fn, lambda cur: cur, cur)

  for i in range(8):
    stgl[pl.ds(cur + i * 16, 16)] = jnp.full((16,), 1, jnp.int32) * dmy
  fire_slabs((cur + 127) >> 7)

  plsc.subcore_barrier()
  pltpu.sync_copy(acc.at[pl.ds(s * RPT, RPT)],
                  out_hbm.at[c, pl.ds(s * RPT, RPT)])


_deg = pl.kernel(
    _deg_body,
    out_type=jax.ShapeDtypeStruct((NC, ACC_R, 16), jnp.float32),
    mesh=_mesh,
    compiler_params=_params,
    scratch_types=[
        pltpu.VMEM_SHARED((ACC_R, 16), jnp.float32),
        pltpu.VMEM((SROWS, 128), jnp.int32),
        pltpu.VMEM((STG,), jnp.int32),
        pltpu.VMEM((128,), jnp.int32),
        pltpu.VMEM((128,), jnp.int32),
        pltpu.VMEM((128, 16), jnp.float32),
        pltpu.VMEM((128, 16), jnp.float32),
        pltpu.SemaphoreType.DMA,
        pltpu.SemaphoreType.DMA,
    ],
)


_BR = 1000


def _dinv_of(deg):
  return jnp.where(deg > 0, lax.rsqrt(jnp.maximum(deg, 1e-12)), 0.0)


def _init_body(x_ref, deg_ref, y_ref, out_ref, d2_ref, d1a_ref):
  x = x_ref[...]
  dinv = _dinv_of(deg_ref[...])
  one = jnp.ones((_BR, D), jnp.float32)
  y_ref[...] = x * dinv
  out_ref[...] = x * ALPHA
  d2_ref[...] = (dinv * dinv) * one
  d1a_ref[...] = (dinv * ALPHA) * one


def _init(x0, deg):
  return pl.pallas_call(
      _init_body,
      grid=(N // _BR,),
      in_specs=[
          pl.BlockSpec((_BR, D), lambda b: (b, 0)),
          pl.BlockSpec((_BR, 1), lambda b: (b, 0)),
      ],
      out_specs=[pl.BlockSpec((_BR, D), lambda b: (b, 0))] * 4,
      out_shape=[jax.ShapeDtypeStruct((N, D), jnp.float32)] * 4,
  )(x0, deg)


def kernel(user_emb, item_emb, edge_index):
  x0 = jnp.concatenate([user_emb, item_emb], axis=0)
  row2 = edge_index[0].astype(jnp.int32).reshape(NRC, 128)
  col2 = edge_index[1].astype(jnp.int32).reshape(NRC, 128)

  deg_planes = _deg(col2)
  deg = jnp.concatenate(
      [deg_planes[0, :HALF, 0], deg_planes[1, :HALF, 0]])[:, None]

  y, out, d2, d1a = _init(x0, deg)
  for layer in range(NUM_LAYERS):
    seg = _segsum_last if layer == NUM_LAYERS - 1 else _segsum
    y, out = seg(y, row2, col2, d2, d1a, out)

  return out[:NU], out[NU:]
